# baseline (speedup 1.0000x reference)
import math

import jax
import jax.numpy as jnp
from jax.experimental import pallas as pl

B, S, IN_DIM = 1, 2048, 512
HID, NH, HD = 512, 16, 32
WIN, TOPK, LR = 5, 10, 32
BS = 256
BA = 1024
QA = S // BA
HPS = 2
QB = S // BS
SPAD = 2056
LS = S + 2 * BS
NEG = -3.0e38


def _conv_body(xpad_ref, wd_ref, cb_ref, h_ref):
    base = pl.program_id(0) * BS
    xs_full = xpad_ref[pl.ds(base, BS + 8), :]
    acc = jnp.zeros((BS, HID), jnp.float32)
    for d in range(7):
        xs = jax.lax.slice(xs_full, (d, 0), (d + BS, IN_DIM))
        acc = acc + jnp.dot(xs, wd_ref[d], preferred_element_type=jnp.float32)
    h_ref[...] = jnp.maximum(acc + cb_ref[...], 0.0)


def _proj_body(h_ref, wq_ref, bq_ref, wk_ref, bk_ref, wv_ref, bv_ref,
               wkp_ref, bkp_ref, wvp_ref, bvp_ref,
               q_ref, kp_ref, vp_ref, kl_ref, vl_ref):
    h = h_ref[...]
    qv = jnp.dot(h, wq_ref[...], preferred_element_type=jnp.float32) + bq_ref[...]
    kv = jnp.dot(h, wk_ref[...], preferred_element_type=jnp.float32) + bk_ref[...]
    vv = jnp.dot(h, wv_ref[...], preferred_element_type=jnp.float32) + bv_ref[...]
    kpv = jnp.dot(h, wkp_ref[...], preferred_element_type=jnp.float32) + bkp_ref[...]
    vpv = jnp.dot(h, wvp_ref[...], preferred_element_type=jnp.float32) + bvp_ref[...]
    for hh in range(NH):
        sl = slice(hh * HD, (hh + 1) * HD)
        q_ref[hh] = qv[:, sl]
        kp_ref[hh] = kpv[:, sl]
        vp_ref[hh] = vpv[:, sl]
        kl_ref[hh] = kv[:, sl]
        vl_ref[hh] = vv[:, sl]


def _attn_body(q_ref, kp_ref, vp_ref, kl_ref, vl_ref, vpw_ref, vpb_ref, o_ref):
    qb = pl.program_id(1)
    base = qb * BA
    for j in range(HPS):
        _attn_one(j, base, q_ref, kp_ref, vp_ref, kl_ref, vl_ref,
                  vpw_ref, vpb_ref, o_ref)


def _attn_one(j, base, q_ref, kp_ref, vp_ref, kl_ref, vl_ref, vpw_ref,
              vpb_ref, o_ref):
    q = q_ref[j]
    kp = kp_ref[j]
    vp = vp_ref[j]

    scores = jax.lax.dot_general(
        q, kp, dimension_numbers=(((1,), (1,)), ((), ())),
        preferred_element_type=jnp.float32)

    a = jax.lax.slice(scores, (0, 0), (BA, S // 2))
    b = jax.lax.slice(scores, (0, S // 2), (BA, S))
    work = jnp.maximum(a, b)
    loser = jnp.minimum(a, b)
    thr = None
    m0 = None
    for i in range(TOPK):
        m = jnp.max(work, axis=1, keepdims=True)
        if i == 0:
            m0 = m
        thr = m
        if i < TOPK - 1:
            hit = work >= m
            work = jnp.where(hit, loser, work)
            if i < TOPK - 2:
                loser = jnp.where(hit, NEG, loser)

    e_all = jnp.exp(scores - m0)
    e = jnp.where(scores >= thr, e_all, 0.0)
    vpx = jnp.concatenate([vp, jnp.ones((S, 8), jnp.float32)], axis=1)
    spx = jnp.dot(e, vpx, preferred_element_type=jnp.float32)
    sp = jax.lax.slice(spx, (0, 0), (BA, LR))
    esum = jax.lax.slice(spx, (0, LR), (BA, LR + 1))
    sp = sp / esum
    sp = jnp.dot(sp, vpw_ref[...], preferred_element_type=jnp.float32) + vpb_ref[...]

    qidx = jax.lax.broadcasted_iota(jnp.int32, (BA, 1), 0) + base
    start = base + BS - 8
    klb = kl_ref[j, pl.ds(start, BA + 16), :]
    vlb = vl_ref[j, pl.ds(start, BA + 16), :]
    ls = []
    valids = []
    for w in range(WIN):
        krows = jax.lax.slice(klb, (w + 6, 0), (w + 6 + BA, HD))
        s_w = jnp.sum(q * krows, axis=1, keepdims=True)
        kpos = qidx + (w - 2)
        valid = (kpos >= 0) & (kpos < S)
        valids.append(valid)
        ls.append(jnp.where(valid, s_w, jnp.float32(-1e30)))
    m_l = ls[0]
    for w in range(1, WIN):
        m_l = jnp.maximum(m_l, ls[w])
    denom = jnp.zeros((BA, 1), jnp.float32)
    acc = jnp.zeros((BA, HD), jnp.float32)
    for w in range(WIN):
        e_w = jnp.exp(ls[w] - m_l)
        denom = denom + e_w
        vrows = jax.lax.slice(vlb, (w + 6, 0), (w + 6 + BA, HD))
        acc = acc + e_w * jnp.where(valids[w], vrows, 0.0)
    lout = acc / denom

    o_ref[j] = (lout + sp) * 0.5


def _ln_body(o_ref, h_ref, ow_ref, ob_ref, g_ref, b_ref, y_ref):
    o2 = jnp.transpose(o_ref[...], (1, 0, 2)).reshape(BS, HID)
    r = jnp.dot(o2, ow_ref[...], preferred_element_type=jnp.float32)
    r = r + ob_ref[...] + h_ref[...]
    mu = jnp.mean(r, axis=1, keepdims=True)
    c = r - mu
    var = jnp.mean(c * c, axis=1, keepdims=True)
    y_ref[...] = c * jax.lax.rsqrt(var + 1e-5) * g_ref[...] + b_ref[...]


@jax.jit
def kernel(x, conv1_w, conv1_b, conv2_w, conv2_b, conv3_w, conv3_b,
           wq, bq, wk, bk, wv, bv, kp_w, kp_b, vp_w, vp_b,
           ow, ob, ln_g, ln_b):
    f32 = jnp.float32
    x2 = x[0]
    xpad = jnp.pad(x2, ((3, 5), (0, 0)))

    wt1 = jnp.pad(jnp.transpose(conv1_w, (2, 1, 0)), ((2, 2), (0, 0), (0, 0)))
    wt2 = jnp.pad(jnp.transpose(conv2_w, (2, 1, 0)), ((1, 1), (0, 0), (0, 0)))
    wt3 = jnp.transpose(conv3_w, (2, 1, 0))
    wd = jnp.concatenate([wt1, wt2, wt3], axis=2)
    cb = jnp.concatenate([conv1_b, conv2_b, conv3_b]).reshape(1, HID)

    h = pl.pallas_call(
        _conv_body,
        grid=(QB,),
        in_specs=[
            pl.BlockSpec((SPAD, IN_DIM), lambda i: (0, 0)),
            pl.BlockSpec((7, IN_DIM, HID), lambda i: (0, 0, 0)),
            pl.BlockSpec((1, HID), lambda i: (0, 0)),
        ],
        out_specs=pl.BlockSpec((BS, HID), lambda i: (i, 0)),
        out_shape=jax.ShapeDtypeStruct((S, HID), f32),
    )(xpad, wd, cb)

    scale = f32(1.0 / math.sqrt(HD))
    eye = jnp.eye(NH, dtype=f32)
    bdk = jnp.kron(eye, kp_w.T)
    bdv = jnp.kron(eye, vp_w.T)
    wkpT = wk.T @ bdk
    bkp = (bk @ bdk + jnp.tile(kp_b, NH)).reshape(1, HID)
    wvpT = wv.T @ bdv
    bvp = (bv @ bdv + jnp.tile(vp_b, NH)).reshape(1, HID)

    full = lambda shape: pl.BlockSpec(shape, lambda i: tuple(0 for _ in shape))
    q, kp, vp, kl, vl = pl.pallas_call(
        _proj_body,
        grid=(QB,),
        in_specs=[
            pl.BlockSpec((BS, HID), lambda i: (i, 0)),
            full((HID, HID)), full((1, HID)),
            full((HID, HID)), full((1, HID)),
            full((HID, HID)), full((1, HID)),
            full((HID, HID)), full((1, HID)),
            full((HID, HID)), full((1, HID)),
        ],
        out_specs=[
            pl.BlockSpec((NH, BS, HD), lambda i: (0, i, 0)),
            pl.BlockSpec((NH, BS, HD), lambda i: (0, i, 0)),
            pl.BlockSpec((NH, BS, HD), lambda i: (0, i, 0)),
            pl.BlockSpec((NH, BS, HD), lambda i: (0, i + 1, 0)),
            pl.BlockSpec((NH, BS, HD), lambda i: (0, i + 1, 0)),
        ],
        out_shape=[
            jax.ShapeDtypeStruct((NH, S, HD), f32),
            jax.ShapeDtypeStruct((NH, S, HD), f32),
            jax.ShapeDtypeStruct((NH, S, HD), f32),
            jax.ShapeDtypeStruct((NH, LS, HD), f32),
            jax.ShapeDtypeStruct((NH, LS, HD), f32),
        ],
    )(h, wq.T * scale, (bq * scale).reshape(1, HID),
      wk.T, bk.reshape(1, HID), wv.T, bv.reshape(1, HID),
      wkpT, bkp, wvpT, bvp)

    o = pl.pallas_call(
        _attn_body,
        grid=(NH // HPS, QA),
        in_specs=[
            pl.BlockSpec((HPS, BA, HD), lambda hh, i: (hh, i, 0)),
            pl.BlockSpec((HPS, S, HD), lambda hh, i: (hh, 0, 0)),
            pl.BlockSpec((HPS, S, HD), lambda hh, i: (hh, 0, 0)),
            pl.BlockSpec((HPS, LS, HD), lambda hh, i: (hh, 0, 0)),
            pl.BlockSpec((HPS, LS, HD), lambda hh, i: (hh, 0, 0)),
            pl.BlockSpec((HD, HD), lambda hh, i: (0, 0)),
            pl.BlockSpec((1, HD), lambda hh, i: (0, 0)),
        ],
        out_specs=pl.BlockSpec((HPS, BA, HD), lambda hh, i: (hh, i, 0)),
        out_shape=jax.ShapeDtypeStruct((NH, S, HD), f32),
    )(q, kp, vp, kl, vl, vp_w.T, vp_b.reshape(1, HD))

    full1 = lambda shape: pl.BlockSpec(shape, lambda i: tuple(0 for _ in shape))
    y = pl.pallas_call(
        _ln_body,
        grid=(QB,),
        in_specs=[
            pl.BlockSpec((NH, BS, HD), lambda i: (0, i, 0)),
            pl.BlockSpec((BS, HID), lambda i: (i, 0)),
            full1((HID, HID)), full1((1, HID)),
            full1((1, HID)), full1((1, HID)),
        ],
        out_specs=pl.BlockSpec((BS, HID), lambda i: (i, 0)),
        out_shape=jax.ShapeDtypeStruct((S, HID), f32),
    )(o, h, ow.T, ob.reshape(1, HID), ln_g.reshape(1, HID), ln_b.reshape(1, HID))

    return y.reshape(B, S, HID)


# speedup vs baseline: 1.0266x; 1.0266x over previous
import math

import jax
import jax.numpy as jnp
from jax.experimental import pallas as pl

B, S, IN_DIM = 1, 2048, 512
HID, NH, HD = 512, 16, 32
WIN, TOPK, LR = 5, 10, 32
BS = 256
BA = 512
QA = S // BA
QB = S // BS
SPAD = 2056
LS = S + 2 * BS
NEG = -3.0e38


def _conv_body(xpad_ref, wd_ref, cb_ref, h_ref):
    base = pl.program_id(0) * BS
    xs_full = xpad_ref[pl.ds(base, BS + 8), :]
    acc = jnp.zeros((BS, HID), jnp.float32)
    for d in range(7):
        xs = jax.lax.slice(xs_full, (d, 0), (d + BS, IN_DIM))
        acc = acc + jnp.dot(xs, wd_ref[d], preferred_element_type=jnp.float32)
    h_ref[...] = jnp.maximum(acc + cb_ref[...], 0.0)


def _proj_body(h_ref, wq_ref, bq_ref, wk_ref, bk_ref, wv_ref, bv_ref,
               wkp_ref, bkp_ref, wvp_ref, bvp_ref,
               q_ref, kp_ref, vp_ref, kl_ref, vl_ref):
    h = h_ref[...]
    qv = jnp.dot(h, wq_ref[...], preferred_element_type=jnp.float32) + bq_ref[...]
    kv = jnp.dot(h, wk_ref[...], preferred_element_type=jnp.float32) + bk_ref[...]
    vv = jnp.dot(h, wv_ref[...], preferred_element_type=jnp.float32) + bv_ref[...]
    kpv = jnp.dot(h, wkp_ref[...], preferred_element_type=jnp.float32) + bkp_ref[...]
    vpv = jnp.dot(h, wvp_ref[...], preferred_element_type=jnp.float32) + bvp_ref[...]
    for hh in range(NH):
        sl = slice(hh * HD, (hh + 1) * HD)
        q_ref[hh] = qv[:, sl]
        kp_ref[hh] = kpv[:, sl]
        vp_ref[hh] = vpv[:, sl]
        kl_ref[hh] = kv[:, sl]
        vl_ref[hh] = vv[:, sl]


def _attn_body(q_ref, kp_ref, vp_ref, kl_ref, vl_ref, vpw_ref, vpb_ref, o_ref):
    qb = pl.program_id(1)
    base = qb * BA
    q = q_ref[0]
    kp = kp_ref[0]
    vp = vp_ref[0]

    scores = jax.lax.dot_general(
        q, kp, dimension_numbers=(((1,), (1,)), ((), ())),
        preferred_element_type=jnp.float32)

    a = jax.lax.slice(scores, (0, 0), (BA, S // 2))
    b = jax.lax.slice(scores, (0, S // 2), (BA, S))
    work = jnp.maximum(a, b)
    loser = jnp.minimum(a, b)
    thr = None
    m0 = None
    for i in range(TOPK):
        m = jnp.max(work, axis=1, keepdims=True)
        if i == 0:
            m0 = m
        thr = m
        if i < TOPK - 1:
            hit = work >= m
            work = jnp.where(hit, loser, work)
            if i < TOPK - 2:
                loser = jnp.where(hit, NEG, loser)

    e_all = jnp.exp(scores - m0)
    e = jnp.where(scores >= thr, e_all, 0.0)
    vpx = jnp.concatenate([vp, jnp.ones((S, 8), jnp.float32)], axis=1)
    spx = jnp.dot(e, vpx, preferred_element_type=jnp.float32)
    sp = jax.lax.slice(spx, (0, 0), (BA, LR))
    esum = jax.lax.slice(spx, (0, LR), (BA, LR + 1))
    sp = sp / esum
    sp = jnp.dot(sp, vpw_ref[...], preferred_element_type=jnp.float32) + vpb_ref[...]

    qidx = jax.lax.broadcasted_iota(jnp.int32, (BA, 1), 0) + base
    start = base + BS - 8
    klb = kl_ref[0, pl.ds(start, BA + 16), :]
    vlb = vl_ref[0, pl.ds(start, BA + 16), :]
    ls = []
    valids = []
    for w in range(WIN):
        krows = jax.lax.slice(klb, (w + 6, 0), (w + 6 + BA, HD))
        s_w = jnp.sum(q * krows, axis=1, keepdims=True)
        kpos = qidx + (w - 2)
        valid = (kpos >= 0) & (kpos < S)
        valids.append(valid)
        ls.append(jnp.where(valid, s_w, jnp.float32(-1e30)))
    m_l = ls[0]
    for w in range(1, WIN):
        m_l = jnp.maximum(m_l, ls[w])
    denom = jnp.zeros((BA, 1), jnp.float32)
    acc = jnp.zeros((BA, HD), jnp.float32)
    for w in range(WIN):
        e_w = jnp.exp(ls[w] - m_l)
        denom = denom + e_w
        vrows = jax.lax.slice(vlb, (w + 6, 0), (w + 6 + BA, HD))
        acc = acc + e_w * jnp.where(valids[w], vrows, 0.0)
    lout = acc / denom

    o_ref[0] = (lout + sp) * 0.5


def _ln_body(o_ref, h_ref, ow_ref, ob_ref, g_ref, b_ref, y_ref):
    o2 = jnp.transpose(o_ref[...], (1, 0, 2)).reshape(BS, HID)
    r = jnp.dot(o2, ow_ref[...], preferred_element_type=jnp.float32)
    r = r + ob_ref[...] + h_ref[...]
    mu = jnp.mean(r, axis=1, keepdims=True)
    c = r - mu
    var = jnp.mean(c * c, axis=1, keepdims=True)
    y_ref[...] = c * jax.lax.rsqrt(var + 1e-5) * g_ref[...] + b_ref[...]


@jax.jit
def kernel(x, conv1_w, conv1_b, conv2_w, conv2_b, conv3_w, conv3_b,
           wq, bq, wk, bk, wv, bv, kp_w, kp_b, vp_w, vp_b,
           ow, ob, ln_g, ln_b):
    f32 = jnp.float32
    x2 = x[0]
    xpad = jnp.pad(x2, ((3, 5), (0, 0)))

    wt1 = jnp.pad(jnp.transpose(conv1_w, (2, 1, 0)), ((2, 2), (0, 0), (0, 0)))
    wt2 = jnp.pad(jnp.transpose(conv2_w, (2, 1, 0)), ((1, 1), (0, 0), (0, 0)))
    wt3 = jnp.transpose(conv3_w, (2, 1, 0))
    wd = jnp.concatenate([wt1, wt2, wt3], axis=2)
    cb = jnp.concatenate([conv1_b, conv2_b, conv3_b]).reshape(1, HID)

    h = pl.pallas_call(
        _conv_body,
        grid=(QB,),
        in_specs=[
            pl.BlockSpec((SPAD, IN_DIM), lambda i: (0, 0)),
            pl.BlockSpec((7, IN_DIM, HID), lambda i: (0, 0, 0)),
            pl.BlockSpec((1, HID), lambda i: (0, 0)),
        ],
        out_specs=pl.BlockSpec((BS, HID), lambda i: (i, 0)),
        out_shape=jax.ShapeDtypeStruct((S, HID), f32),
    )(xpad, wd, cb)

    scale = f32(1.0 / math.sqrt(HD))
    eye = jnp.eye(NH, dtype=f32)
    bdk = jnp.kron(eye, kp_w.T)
    bdv = jnp.kron(eye, vp_w.T)
    wkpT = wk.T @ bdk
    bkp = (bk @ bdk + jnp.tile(kp_b, NH)).reshape(1, HID)
    wvpT = wv.T @ bdv
    bvp = (bv @ bdv + jnp.tile(vp_b, NH)).reshape(1, HID)

    full = lambda shape: pl.BlockSpec(shape, lambda i: tuple(0 for _ in shape))
    q, kp, vp, kl, vl = pl.pallas_call(
        _proj_body,
        grid=(QB,),
        in_specs=[
            pl.BlockSpec((BS, HID), lambda i: (i, 0)),
            full((HID, HID)), full((1, HID)),
            full((HID, HID)), full((1, HID)),
            full((HID, HID)), full((1, HID)),
            full((HID, HID)), full((1, HID)),
            full((HID, HID)), full((1, HID)),
        ],
        out_specs=[
            pl.BlockSpec((NH, BS, HD), lambda i: (0, i, 0)),
            pl.BlockSpec((NH, BS, HD), lambda i: (0, i, 0)),
            pl.BlockSpec((NH, BS, HD), lambda i: (0, i, 0)),
            pl.BlockSpec((NH, BS, HD), lambda i: (0, i + 1, 0)),
            pl.BlockSpec((NH, BS, HD), lambda i: (0, i + 1, 0)),
        ],
        out_shape=[
            jax.ShapeDtypeStruct((NH, S, HD), f32),
            jax.ShapeDtypeStruct((NH, S, HD), f32),
            jax.ShapeDtypeStruct((NH, S, HD), f32),
            jax.ShapeDtypeStruct((NH, LS, HD), f32),
            jax.ShapeDtypeStruct((NH, LS, HD), f32),
        ],
    )(h, wq.T * scale, (bq * scale).reshape(1, HID),
      wk.T, bk.reshape(1, HID), wv.T, bv.reshape(1, HID),
      wkpT, bkp, wvpT, bvp)

    o = pl.pallas_call(
        _attn_body,
        grid=(NH, QA),
        in_specs=[
            pl.BlockSpec((1, BA, HD), lambda hh, i: (hh, i, 0)),
            pl.BlockSpec((1, S, HD), lambda hh, i: (hh, 0, 0)),
            pl.BlockSpec((1, S, HD), lambda hh, i: (hh, 0, 0)),
            pl.BlockSpec((1, LS, HD), lambda hh, i: (hh, 0, 0)),
            pl.BlockSpec((1, LS, HD), lambda hh, i: (hh, 0, 0)),
            pl.BlockSpec((HD, HD), lambda hh, i: (0, 0)),
            pl.BlockSpec((1, HD), lambda hh, i: (0, 0)),
        ],
        out_specs=pl.BlockSpec((1, BA, HD), lambda hh, i: (hh, i, 0)),
        out_shape=jax.ShapeDtypeStruct((NH, S, HD), f32),
    )(q, kp, vp, kl, vl, vp_w.T, vp_b.reshape(1, HD))

    full1 = lambda shape: pl.BlockSpec(shape, lambda i: tuple(0 for _ in shape))
    y = pl.pallas_call(
        _ln_body,
        grid=(QB,),
        in_specs=[
            pl.BlockSpec((NH, BS, HD), lambda i: (0, i, 0)),
            pl.BlockSpec((BS, HID), lambda i: (i, 0)),
            full1((HID, HID)), full1((1, HID)),
            full1((1, HID)), full1((1, HID)),
        ],
        out_specs=pl.BlockSpec((BS, HID), lambda i: (i, 0)),
        out_shape=jax.ShapeDtypeStruct((S, HID), f32),
    )(o, h, ow.T, ob.reshape(1, HID), ln_g.reshape(1, HID), ln_b.reshape(1, HID))

    return y.reshape(B, S, HID)


# speedup vs baseline: 1.0624x; 1.0349x over previous
import math

import jax
import jax.numpy as jnp
from jax.experimental import pallas as pl

B, S, IN_DIM = 1, 2048, 512
HID, NH, HD = 512, 16, 32
WIN, TOPK, LR = 5, 10, 32
BS = 256
BA = 2048
QA = S // BA
QB = S // BS
SPAD = 2056
LS = S + 2 * BS
NEG = -3.0e38


def _conv_body(xpad_ref, wd_ref, cb_ref, h_ref):
    base = pl.program_id(0) * BS
    xs_full = xpad_ref[pl.ds(base, BS + 8), :]
    acc = jnp.zeros((BS, HID), jnp.float32)
    for d in range(7):
        xs = jax.lax.slice(xs_full, (d, 0), (d + BS, IN_DIM))
        acc = acc + jnp.dot(xs, wd_ref[d], preferred_element_type=jnp.float32)
    h_ref[...] = jnp.maximum(acc + cb_ref[...], 0.0)


def _proj_body(h_ref, wq_ref, bq_ref, wk_ref, bk_ref, wv_ref, bv_ref,
               wkp_ref, bkp_ref, wvp_ref, bvp_ref,
               q_ref, kp_ref, vp_ref, kl_ref, vl_ref):
    h = h_ref[...]
    qv = jnp.dot(h, wq_ref[...], preferred_element_type=jnp.float32) + bq_ref[...]
    kv = jnp.dot(h, wk_ref[...], preferred_element_type=jnp.float32) + bk_ref[...]
    vv = jnp.dot(h, wv_ref[...], preferred_element_type=jnp.float32) + bv_ref[...]
    kpv = jnp.dot(h, wkp_ref[...], preferred_element_type=jnp.float32) + bkp_ref[...]
    vpv = jnp.dot(h, wvp_ref[...], preferred_element_type=jnp.float32) + bvp_ref[...]
    for hh in range(NH):
        sl = slice(hh * HD, (hh + 1) * HD)
        q_ref[hh] = qv[:, sl]
        kp_ref[hh] = kpv[:, sl]
        vp_ref[hh] = vpv[:, sl]
        kl_ref[hh] = kv[:, sl]
        vl_ref[hh] = vv[:, sl]


def _attn_body(q_ref, kp_ref, vp_ref, kl_ref, vl_ref, vpw_ref, vpb_ref, o_ref):
    qb = pl.program_id(1)
    base = qb * BA
    q = q_ref[0]
    kp = kp_ref[0]
    vp = vp_ref[0]

    scores = jax.lax.dot_general(
        q, kp, dimension_numbers=(((1,), (1,)), ((), ())),
        preferred_element_type=jnp.float32)

    a = jax.lax.slice(scores, (0, 0), (BA, S // 2))
    b = jax.lax.slice(scores, (0, S // 2), (BA, S))
    work = jnp.maximum(a, b)
    loser = jnp.minimum(a, b)
    thr = None
    m0 = None
    for i in range(TOPK):
        m = jnp.max(work, axis=1, keepdims=True)
        if i == 0:
            m0 = m
        thr = m
        if i < TOPK - 1:
            hit = work >= m
            work = jnp.where(hit, loser, work)
            if i < TOPK - 2:
                loser = jnp.where(hit, NEG, loser)

    e_all = jnp.exp(scores - m0)
    e = jnp.where(scores >= thr, e_all, 0.0)
    vpx = jnp.concatenate([vp, jnp.ones((S, 8), jnp.float32)], axis=1)
    spx = jnp.dot(e, vpx, preferred_element_type=jnp.float32)
    sp = jax.lax.slice(spx, (0, 0), (BA, LR))
    esum = jax.lax.slice(spx, (0, LR), (BA, LR + 1))
    sp = sp / esum
    sp = jnp.dot(sp, vpw_ref[...], preferred_element_type=jnp.float32) + vpb_ref[...]

    qidx = jax.lax.broadcasted_iota(jnp.int32, (BA, 1), 0) + base
    start = base + BS - 8
    klb = kl_ref[0, pl.ds(start, BA + 16), :]
    vlb = vl_ref[0, pl.ds(start, BA + 16), :]
    ls = []
    valids = []
    for w in range(WIN):
        krows = jax.lax.slice(klb, (w + 6, 0), (w + 6 + BA, HD))
        s_w = jnp.sum(q * krows, axis=1, keepdims=True)
        kpos = qidx + (w - 2)
        valid = (kpos >= 0) & (kpos < S)
        valids.append(valid)
        ls.append(jnp.where(valid, s_w, jnp.float32(-1e30)))
    m_l = ls[0]
    for w in range(1, WIN):
        m_l = jnp.maximum(m_l, ls[w])
    denom = jnp.zeros((BA, 1), jnp.float32)
    acc = jnp.zeros((BA, HD), jnp.float32)
    for w in range(WIN):
        e_w = jnp.exp(ls[w] - m_l)
        denom = denom + e_w
        vrows = jax.lax.slice(vlb, (w + 6, 0), (w + 6 + BA, HD))
        acc = acc + e_w * jnp.where(valids[w], vrows, 0.0)
    lout = acc / denom

    o_ref[0] = (lout + sp) * 0.5


def _ln_body(o_ref, h_ref, ow_ref, ob_ref, g_ref, b_ref, y_ref):
    o2 = jnp.transpose(o_ref[...], (1, 0, 2)).reshape(BS, HID)
    r = jnp.dot(o2, ow_ref[...], preferred_element_type=jnp.float32)
    r = r + ob_ref[...] + h_ref[...]
    mu = jnp.mean(r, axis=1, keepdims=True)
    c = r - mu
    var = jnp.mean(c * c, axis=1, keepdims=True)
    y_ref[...] = c * jax.lax.rsqrt(var + 1e-5) * g_ref[...] + b_ref[...]


@jax.jit
def kernel(x, conv1_w, conv1_b, conv2_w, conv2_b, conv3_w, conv3_b,
           wq, bq, wk, bk, wv, bv, kp_w, kp_b, vp_w, vp_b,
           ow, ob, ln_g, ln_b):
    f32 = jnp.float32
    x2 = x[0]
    xpad = jnp.pad(x2, ((3, 5), (0, 0)))

    wt1 = jnp.pad(jnp.transpose(conv1_w, (2, 1, 0)), ((2, 2), (0, 0), (0, 0)))
    wt2 = jnp.pad(jnp.transpose(conv2_w, (2, 1, 0)), ((1, 1), (0, 0), (0, 0)))
    wt3 = jnp.transpose(conv3_w, (2, 1, 0))
    wd = jnp.concatenate([wt1, wt2, wt3], axis=2)
    cb = jnp.concatenate([conv1_b, conv2_b, conv3_b]).reshape(1, HID)

    h = pl.pallas_call(
        _conv_body,
        grid=(QB,),
        in_specs=[
            pl.BlockSpec((SPAD, IN_DIM), lambda i: (0, 0)),
            pl.BlockSpec((7, IN_DIM, HID), lambda i: (0, 0, 0)),
            pl.BlockSpec((1, HID), lambda i: (0, 0)),
        ],
        out_specs=pl.BlockSpec((BS, HID), lambda i: (i, 0)),
        out_shape=jax.ShapeDtypeStruct((S, HID), f32),
    )(xpad, wd, cb)

    scale = f32(1.0 / math.sqrt(HD))
    eye = jnp.eye(NH, dtype=f32)
    bdk = jnp.kron(eye, kp_w.T)
    bdv = jnp.kron(eye, vp_w.T)
    wkpT = wk.T @ bdk
    bkp = (bk @ bdk + jnp.tile(kp_b, NH)).reshape(1, HID)
    wvpT = wv.T @ bdv
    bvp = (bv @ bdv + jnp.tile(vp_b, NH)).reshape(1, HID)

    full = lambda shape: pl.BlockSpec(shape, lambda i: tuple(0 for _ in shape))
    q, kp, vp, kl, vl = pl.pallas_call(
        _proj_body,
        grid=(QB,),
        in_specs=[
            pl.BlockSpec((BS, HID), lambda i: (i, 0)),
            full((HID, HID)), full((1, HID)),
            full((HID, HID)), full((1, HID)),
            full((HID, HID)), full((1, HID)),
            full((HID, HID)), full((1, HID)),
            full((HID, HID)), full((1, HID)),
        ],
        out_specs=[
            pl.BlockSpec((NH, BS, HD), lambda i: (0, i, 0)),
            pl.BlockSpec((NH, BS, HD), lambda i: (0, i, 0)),
            pl.BlockSpec((NH, BS, HD), lambda i: (0, i, 0)),
            pl.BlockSpec((NH, BS, HD), lambda i: (0, i + 1, 0)),
            pl.BlockSpec((NH, BS, HD), lambda i: (0, i + 1, 0)),
        ],
        out_shape=[
            jax.ShapeDtypeStruct((NH, S, HD), f32),
            jax.ShapeDtypeStruct((NH, S, HD), f32),
            jax.ShapeDtypeStruct((NH, S, HD), f32),
            jax.ShapeDtypeStruct((NH, LS, HD), f32),
            jax.ShapeDtypeStruct((NH, LS, HD), f32),
        ],
    )(h, wq.T * scale, (bq * scale).reshape(1, HID),
      wk.T, bk.reshape(1, HID), wv.T, bv.reshape(1, HID),
      wkpT, bkp, wvpT, bvp)

    o = pl.pallas_call(
        _attn_body,
        grid=(NH, QA),
        in_specs=[
            pl.BlockSpec((1, BA, HD), lambda hh, i: (hh, i, 0)),
            pl.BlockSpec((1, S, HD), lambda hh, i: (hh, 0, 0)),
            pl.BlockSpec((1, S, HD), lambda hh, i: (hh, 0, 0)),
            pl.BlockSpec((1, LS, HD), lambda hh, i: (hh, 0, 0)),
            pl.BlockSpec((1, LS, HD), lambda hh, i: (hh, 0, 0)),
            pl.BlockSpec((HD, HD), lambda hh, i: (0, 0)),
            pl.BlockSpec((1, HD), lambda hh, i: (0, 0)),
        ],
        out_specs=pl.BlockSpec((1, BA, HD), lambda hh, i: (hh, i, 0)),
        out_shape=jax.ShapeDtypeStruct((NH, S, HD), f32),
    )(q, kp, vp, kl, vl, vp_w.T, vp_b.reshape(1, HD))

    full1 = lambda shape: pl.BlockSpec(shape, lambda i: tuple(0 for _ in shape))
    y = pl.pallas_call(
        _ln_body,
        grid=(QB,),
        in_specs=[
            pl.BlockSpec((NH, BS, HD), lambda i: (0, i, 0)),
            pl.BlockSpec((BS, HID), lambda i: (i, 0)),
            full1((HID, HID)), full1((1, HID)),
            full1((1, HID)), full1((1, HID)),
        ],
        out_specs=pl.BlockSpec((BS, HID), lambda i: (i, 0)),
        out_shape=jax.ShapeDtypeStruct((S, HID), f32),
    )(o, h, ow.T, ob.reshape(1, HID), ln_g.reshape(1, HID), ln_b.reshape(1, HID))

    return y.reshape(B, S, HID)


# speedup vs baseline: 1.0769x; 1.0137x over previous
import math

import jax
import jax.numpy as jnp
from jax.experimental import pallas as pl

B, S, IN_DIM = 1, 2048, 512
HID, NH, HD = 512, 16, 32
WIN, TOPK, LR = 5, 10, 32
BS = 256
BA = 2048
QA = S // BA
QB = S // BS
SPAD = 2056
LS = S + 2 * BS
NEG = -3.0e38


def _front_body(xpad_ref, wd_ref, cb_ref, wq_ref, bq_ref, wk_ref, bk_ref,
                wv_ref, bv_ref, wkp_ref, bkp_ref, wvp_ref, bvp_ref,
                h_ref, q_ref, kp_ref, vp_ref, kl_ref, vl_ref):
    base = pl.program_id(0) * BS
    xs_full = xpad_ref[pl.ds(base, BS + 8), :]
    acc = jnp.zeros((BS, HID), jnp.float32)
    for d in range(7):
        xs = jax.lax.slice(xs_full, (d, 0), (d + BS, IN_DIM))
        acc = acc + jnp.dot(xs, wd_ref[d], preferred_element_type=jnp.float32)
    h = jnp.maximum(acc + cb_ref[...], 0.0)
    h_ref[...] = h
    qv = jnp.dot(h, wq_ref[...], preferred_element_type=jnp.float32) + bq_ref[...]
    kv = jnp.dot(h, wk_ref[...], preferred_element_type=jnp.float32) + bk_ref[...]
    vv = jnp.dot(h, wv_ref[...], preferred_element_type=jnp.float32) + bv_ref[...]
    kpv = jnp.dot(h, wkp_ref[...], preferred_element_type=jnp.float32) + bkp_ref[...]
    vpv = jnp.dot(h, wvp_ref[...], preferred_element_type=jnp.float32) + bvp_ref[...]
    for hh in range(NH):
        sl = slice(hh * HD, (hh + 1) * HD)
        q_ref[hh] = qv[:, sl]
        kp_ref[hh] = kpv[:, sl]
        vp_ref[hh] = vpv[:, sl]
        kl_ref[hh] = kv[:, sl]
        vl_ref[hh] = vv[:, sl]


def _attn_body(q_ref, kp_ref, vp_ref, kl_ref, vl_ref, vpw_ref, vpb_ref, o_ref):
    qb = pl.program_id(1)
    base = qb * BA
    q = q_ref[0]
    kp = kp_ref[0]
    vp = vp_ref[0]

    scores = jax.lax.dot_general(
        q, kp, dimension_numbers=(((1,), (1,)), ((), ())),
        preferred_element_type=jnp.float32)

    a = jax.lax.slice(scores, (0, 0), (BA, S // 2))
    b = jax.lax.slice(scores, (0, S // 2), (BA, S))
    work = jnp.maximum(a, b)
    loser = jnp.minimum(a, b)
    thr = None
    m0 = None
    for i in range(TOPK):
        m = jnp.max(work, axis=1, keepdims=True)
        if i == 0:
            m0 = m
        thr = m
        if i < TOPK - 1:
            hit = work >= m
            work = jnp.where(hit, loser, work)
            if i < TOPK - 2:
                loser = jnp.where(hit, NEG, loser)

    e_all = jnp.exp(scores - m0)
    e = jnp.where(scores >= thr, e_all, 0.0)
    vpx = jnp.concatenate([vp, jnp.ones((S, 8), jnp.float32)], axis=1)
    spx = jnp.dot(e, vpx, preferred_element_type=jnp.float32)
    sp = jax.lax.slice(spx, (0, 0), (BA, LR))
    esum = jax.lax.slice(spx, (0, LR), (BA, LR + 1))
    sp = sp / esum
    sp = jnp.dot(sp, vpw_ref[...], preferred_element_type=jnp.float32) + vpb_ref[...]

    qidx = jax.lax.broadcasted_iota(jnp.int32, (BA, 1), 0) + base
    start = base + BS - 8
    klb = kl_ref[0, pl.ds(start, BA + 16), :]
    vlb = vl_ref[0, pl.ds(start, BA + 16), :]
    ls = []
    valids = []
    for w in range(WIN):
        krows = jax.lax.slice(klb, (w + 6, 0), (w + 6 + BA, HD))
        s_w = jnp.sum(q * krows, axis=1, keepdims=True)
        kpos = qidx + (w - 2)
        valid = (kpos >= 0) & (kpos < S)
        valids.append(valid)
        ls.append(jnp.where(valid, s_w, jnp.float32(-1e30)))
    m_l = ls[0]
    for w in range(1, WIN):
        m_l = jnp.maximum(m_l, ls[w])
    denom = jnp.zeros((BA, 1), jnp.float32)
    acc = jnp.zeros((BA, HD), jnp.float32)
    for w in range(WIN):
        e_w = jnp.exp(ls[w] - m_l)
        denom = denom + e_w
        vrows = jax.lax.slice(vlb, (w + 6, 0), (w + 6 + BA, HD))
        acc = acc + e_w * jnp.where(valids[w], vrows, 0.0)
    lout = acc / denom

    o_ref[0] = (lout + sp) * 0.5


def _ln_body(o_ref, h_ref, ow_ref, ob_ref, g_ref, b_ref, y_ref):
    o2 = jnp.transpose(o_ref[...], (1, 0, 2)).reshape(BS, HID)
    r = jnp.dot(o2, ow_ref[...], preferred_element_type=jnp.float32)
    r = r + ob_ref[...] + h_ref[...]
    mu = jnp.mean(r, axis=1, keepdims=True)
    c = r - mu
    var = jnp.mean(c * c, axis=1, keepdims=True)
    y_ref[...] = c * jax.lax.rsqrt(var + 1e-5) * g_ref[...] + b_ref[...]


@jax.jit
def kernel(x, conv1_w, conv1_b, conv2_w, conv2_b, conv3_w, conv3_b,
           wq, bq, wk, bk, wv, bv, kp_w, kp_b, vp_w, vp_b,
           ow, ob, ln_g, ln_b):
    f32 = jnp.float32
    x2 = x[0]
    xpad = jnp.pad(x2, ((3, 5), (0, 0)))

    wt1 = jnp.pad(jnp.transpose(conv1_w, (2, 1, 0)), ((2, 2), (0, 0), (0, 0)))
    wt2 = jnp.pad(jnp.transpose(conv2_w, (2, 1, 0)), ((1, 1), (0, 0), (0, 0)))
    wt3 = jnp.transpose(conv3_w, (2, 1, 0))
    wd = jnp.concatenate([wt1, wt2, wt3], axis=2)
    cb = jnp.concatenate([conv1_b, conv2_b, conv3_b]).reshape(1, HID)

    scale = f32(1.0 / math.sqrt(HD))
    eye = jnp.eye(NH, dtype=f32)
    bdk = jnp.kron(eye, kp_w.T)
    bdv = jnp.kron(eye, vp_w.T)
    wkpT = wk.T @ bdk
    bkp = (bk @ bdk + jnp.tile(kp_b, NH)).reshape(1, HID)
    wvpT = wv.T @ bdv
    bvp = (bv @ bdv + jnp.tile(vp_b, NH)).reshape(1, HID)

    full = lambda shape: pl.BlockSpec(shape, lambda i: tuple(0 for _ in shape))
    h, q, kp, vp, kl, vl = pl.pallas_call(
        _front_body,
        grid=(QB,),
        in_specs=[
            pl.BlockSpec((SPAD, IN_DIM), lambda i: (0, 0)),
            pl.BlockSpec((7, IN_DIM, HID), lambda i: (0, 0, 0)),
            full((1, HID)),
            full((HID, HID)), full((1, HID)),
            full((HID, HID)), full((1, HID)),
            full((HID, HID)), full((1, HID)),
            full((HID, HID)), full((1, HID)),
            full((HID, HID)), full((1, HID)),
        ],
        out_specs=[
            pl.BlockSpec((BS, HID), lambda i: (i, 0)),
            pl.BlockSpec((NH, BS, HD), lambda i: (0, i, 0)),
            pl.BlockSpec((NH, BS, HD), lambda i: (0, i, 0)),
            pl.BlockSpec((NH, BS, HD), lambda i: (0, i, 0)),
            pl.BlockSpec((NH, BS, HD), lambda i: (0, i + 1, 0)),
            pl.BlockSpec((NH, BS, HD), lambda i: (0, i + 1, 0)),
        ],
        out_shape=[
            jax.ShapeDtypeStruct((S, HID), f32),
            jax.ShapeDtypeStruct((NH, S, HD), f32),
            jax.ShapeDtypeStruct((NH, S, HD), f32),
            jax.ShapeDtypeStruct((NH, S, HD), f32),
            jax.ShapeDtypeStruct((NH, LS, HD), f32),
            jax.ShapeDtypeStruct((NH, LS, HD), f32),
        ],
    )(xpad, wd, cb, wq.T * scale, (bq * scale).reshape(1, HID),
      wk.T, bk.reshape(1, HID), wv.T, bv.reshape(1, HID),
      wkpT, bkp, wvpT, bvp)

    o = pl.pallas_call(
        _attn_body,
        grid=(NH, QA),
        in_specs=[
            pl.BlockSpec((1, BA, HD), lambda hh, i: (hh, i, 0)),
            pl.BlockSpec((1, S, HD), lambda hh, i: (hh, 0, 0)),
            pl.BlockSpec((1, S, HD), lambda hh, i: (hh, 0, 0)),
            pl.BlockSpec((1, LS, HD), lambda hh, i: (hh, 0, 0)),
            pl.BlockSpec((1, LS, HD), lambda hh, i: (hh, 0, 0)),
            pl.BlockSpec((HD, HD), lambda hh, i: (0, 0)),
            pl.BlockSpec((1, HD), lambda hh, i: (0, 0)),
        ],
        out_specs=pl.BlockSpec((1, BA, HD), lambda hh, i: (hh, i, 0)),
        out_shape=jax.ShapeDtypeStruct((NH, S, HD), f32),
    )(q, kp, vp, kl, vl, vp_w.T, vp_b.reshape(1, HD))

    full1 = lambda shape: pl.BlockSpec(shape, lambda i: tuple(0 for _ in shape))
    y = pl.pallas_call(
        _ln_body,
        grid=(QB,),
        in_specs=[
            pl.BlockSpec((NH, BS, HD), lambda i: (0, i, 0)),
            pl.BlockSpec((BS, HID), lambda i: (i, 0)),
            full1((HID, HID)), full1((1, HID)),
            full1((1, HID)), full1((1, HID)),
        ],
        out_specs=pl.BlockSpec((BS, HID), lambda i: (i, 0)),
        out_shape=jax.ShapeDtypeStruct((S, HID), f32),
    )(o, h, ow.T, ob.reshape(1, HID), ln_g.reshape(1, HID), ln_b.reshape(1, HID))

    return y.reshape(B, S, HID)
